# baseline (speedup 1.0000x reference)
import functools

import jax
import jax.numpy as jnp
from jax import lax
from jax.experimental import pallas as pl
from jax.experimental.pallas import tpu as pltpu
from jax.experimental.pallas import tpu_sc as plsc

_info = plsc.get_sparse_core_info()
_NC, _NS = _info.num_cores, _info.num_subcores
_NW = _NC * _NS


def _make_sc_gather(V, D, N, chunk=400, nbuf=2):
    assert N % _NW == 0
    rows_per_w = N // _NW
    assert rows_per_w % chunk == 0
    nchunks = rows_per_w // chunk
    mesh = plsc.VectorSubcoreMesh(core_axis_name="c", subcore_axis_name="s")

    @functools.partial(
        pl.kernel,
        mesh=mesh,
        out_type=jax.ShapeDtypeStruct((N, D), jnp.float32),
        scratch_types=[
            pltpu.VMEM((rows_per_w,), jnp.int32),
            pltpu.VMEM((nbuf, chunk, D), jnp.float32),
        ] + [pltpu.SemaphoreType.DMA] * nbuf,
    )
    def gather_kernel(table_hbm, idx_hbm, out_hbm, idx_v, rows_v, *sems):
        wid = lax.axis_index("s") * _NC + lax.axis_index("c")
        base = wid * rows_per_w
        pltpu.sync_copy(idx_hbm.at[pl.ds(base, rows_per_w)], idx_v)
        handles = [None] * nchunks
        for g in range(nbuf):
            handles[g] = pltpu.async_copy(
                table_hbm.at[idx_v.at[pl.ds(g * chunk, chunk)]],
                rows_v.at[g % nbuf], sems[g % nbuf])
        for g in range(nchunks):
            handles[g].wait()
            pltpu.sync_copy(rows_v.at[g % nbuf],
                            out_hbm.at[pl.ds(base + g * chunk, chunk)])
            nxt = g + nbuf
            if nxt < nchunks:
                handles[nxt] = pltpu.async_copy(
                    table_hbm.at[idx_v.at[pl.ds(nxt * chunk, chunk)]],
                    rows_v.at[nxt % nbuf], sems[nxt % nbuf])

    return gather_kernel


def _tc_body(tok_ref, cnn_ref, pos_ref, w_ref, b_ref, gamma_ref, beta_ref,
             out_ref):
    tok = tok_ref[...]
    cnn = cnn_ref[...]
    bb, seq, d = tok.shape
    dense = jnp.dot(cnn.reshape(bb * seq, cnn.shape[-1]), w_ref[...],
                    preferred_element_type=jnp.float32).reshape(bb, seq, d)
    comb = tok + dense + pos_ref[...][None] + b_ref[...][None]
    mean = jnp.mean(comb, axis=-1, keepdims=True)
    cent = comb - mean
    var = jnp.mean(cent * cent, axis=-1, keepdims=True)
    normed = cent * lax.rsqrt(var + 1e-5)
    out_ref[...] = normed * gamma_ref[...][None] + beta_ref[...][None]


def kernel(x, cnn_features, tok_table, pos_table, W, b, gamma, beta):
    B, L = x.shape
    V, D = tok_table.shape
    CD = cnn_features.shape[-1]

    CH = 4
    BC = B // CH
    NC = BC * L
    sc_gather = _make_sc_gather(V, D, NC)
    BB = 8
    tc_call = pl.pallas_call(
        _tc_body,
        grid=(BC // BB,),
        in_specs=[
            pl.BlockSpec((BB, L, D), lambda i: (i, 0, 0)),
            pl.BlockSpec((BB, L, CD), lambda i: (i, 0, 0)),
            pl.BlockSpec((L, D), lambda i: (0, 0)),
            pl.BlockSpec((CD, D), lambda i: (0, 0)),
            pl.BlockSpec((1, D), lambda i: (0, 0)),
            pl.BlockSpec((1, D), lambda i: (0, 0)),
            pl.BlockSpec((1, D), lambda i: (0, 0)),
        ],
        out_specs=pl.BlockSpec((BB, L, D), lambda i: (i, 0, 0)),
        out_shape=jax.ShapeDtypeStruct((BC, L, D), jnp.float32),
    )
    b2, g2, be2 = b.reshape(1, D), gamma.reshape(1, D), beta.reshape(1, D)
    outs = []
    for c in range(CH):
        gathered = sc_gather(tok_table, x[c * BC:(c + 1) * BC].reshape(NC))
        outs.append(tc_call(gathered.reshape(BC, L, D),
                            cnn_features[c * BC:(c + 1) * BC],
                            pos_table, W, b2, g2, be2))
    return jnp.concatenate(outs, axis=0)


# speedup vs baseline: 1.9301x; 1.9301x over previous
import functools

import jax
import jax.numpy as jnp
from jax import lax
from jax.experimental import pallas as pl
from jax.experimental.pallas import tpu as pltpu
from jax.experimental.pallas import tpu_sc as plsc

_info = plsc.get_sparse_core_info()
_NC, _NS = _info.num_cores, _info.num_subcores
_NW = _NC * _NS


def _make_sc_gather_pack(V, D, N, chunk=160, nbuf=2):
    assert N % _NW == 0
    rows_per_w = N // _NW
    half = rows_per_w // 2
    assert half % chunk == 0
    npairs = half // chunk
    mesh = plsc.VectorSubcoreMesh(core_axis_name="c", subcore_axis_name="s")

    @functools.partial(
        pl.kernel,
        mesh=mesh,
        out_type=jax.ShapeDtypeStruct((N // 2, D), jnp.int32),
        scratch_types=[
            pltpu.VMEM((rows_per_w,), jnp.int32),
            pltpu.VMEM((nbuf, 2, chunk, D), jnp.float32),
            pltpu.VMEM((chunk, D), jnp.int32),
        ] + [pltpu.SemaphoreType.DMA] * (2 * nbuf),
    )
    def gather_kernel(table_hbm, idx_hbm, out_hbm, idx_v, rows_v, packed_v,
                      *sems):
        wid = lax.axis_index("s") * _NC + lax.axis_index("c")
        base = wid * rows_per_w
        pltpu.sync_copy(idx_hbm.at[pl.ds(base, rows_per_w)], idx_v)

        def fire(g):
            slot = g % nbuf
            ha = pltpu.async_copy(
                table_hbm.at[idx_v.at[pl.ds(g * chunk, chunk)]],
                rows_v.at[slot, 0], sems[2 * slot])
            hb = pltpu.async_copy(
                table_hbm.at[idx_v.at[pl.ds(half + g * chunk, chunk)]],
                rows_v.at[slot, 1], sems[2 * slot + 1])
            return ha, hb

        handles = [None] * npairs
        for g in range(nbuf):
            handles[g] = fire(g)
        for g in range(npairs):
            handles[g][0].wait()
            handles[g][1].wait()
            slot = g % nbuf

            @plsc.parallel_loop(0, chunk)
            def _conv(r):
                for k in range(D // 16):
                    a = rows_v[slot, 0, r, pl.ds(k * 16, 16)]
                    c = rows_v[slot, 1, r, pl.ds(k * 16, 16)]
                    ai = lax.bitcast_convert_type(a, jnp.int32)
                    ci = lax.bitcast_convert_type(c, jnp.int32)
                    ar = ai + 32767 + ((ai >> 16) & 1)
                    cr = ci + 32767 + ((ci >> 16) & 1)
                    packed_v[r, pl.ds(k * 16, 16)] = (
                        ((ar >> 16) & 65535) | (cr & -65536))

            pltpu.sync_copy(
                packed_v,
                out_hbm.at[pl.ds(wid * half + g * chunk, chunk)])
            nxt = g + nbuf
            if nxt < npairs:
                handles[nxt] = fire(nxt)

    return gather_kernel


def _tc_body(tokp_ref, cnn_ref, pos_ref, w_ref, b_ref, gamma_ref, beta_ref,
             out_ref):
    xi = tokp_ref[...][0]
    lo = lax.bitcast_convert_type(xi << 16, jnp.float32)
    hi = lax.bitcast_convert_type(xi & jnp.int32(-65536), jnp.float32)
    cnn = cnn_ref[...]
    bb, seq, cd = cnn.shape
    d = xi.shape[-1]
    tok = jnp.concatenate([lo, hi], axis=0).reshape(bb, seq, d)
    dense = jnp.dot(cnn.reshape(bb * seq, cd), w_ref[...],
                    preferred_element_type=jnp.float32).reshape(bb, seq, d)
    comb = tok + dense + pos_ref[...][None] + b_ref[...][None]
    mean = jnp.mean(comb, axis=-1, keepdims=True)
    cent = comb - mean
    var = jnp.mean(cent * cent, axis=-1, keepdims=True)
    normed = cent * lax.rsqrt(var + 1e-5)
    out_ref[...] = normed * gamma_ref[...][None] + beta_ref[...][None]


def kernel(x, cnn_features, tok_table, pos_table, W, b, gamma, beta):
    B, L = x.shape
    V, D = tok_table.shape
    CD = cnn_features.shape[-1]
    N = B * L

    packed = _make_sc_gather_pack(V, D, N)(tok_table, x.reshape(N))
    H = N // 2 // _NW
    packed = packed.reshape(_NW, H, D)

    BB = B // _NW
    out = pl.pallas_call(
        _tc_body,
        grid=(_NW,),
        in_specs=[
            pl.BlockSpec((1, H, D), lambda i: (i, 0, 0)),
            pl.BlockSpec((BB, L, CD), lambda i: (i, 0, 0)),
            pl.BlockSpec((L, D), lambda i: (0, 0)),
            pl.BlockSpec((CD, D), lambda i: (0, 0)),
            pl.BlockSpec((1, D), lambda i: (0, 0)),
            pl.BlockSpec((1, D), lambda i: (0, 0)),
            pl.BlockSpec((1, D), lambda i: (0, 0)),
        ],
        out_specs=pl.BlockSpec((BB, L, D), lambda i: (i, 0, 0)),
        out_shape=jax.ShapeDtypeStruct((B, L, D), jnp.float32),
    )(packed, cnn_features, pos_table, W, b.reshape(1, D),
      gamma.reshape(1, D), beta.reshape(1, D))
    return out
